# Initial kernel scaffold; baseline (speedup 1.0000x reference)
import jax
import jax.numpy as jnp
from jax.experimental import pallas as pl


def kernel(x, edge_index, eps, W1, b1, g1, be1, W2, b2, g2, be2):
    raise NotImplementedError("write your pallas kernel here")


# speedup vs baseline: 6.5585x; 6.5585x over previous
import functools

import jax
import jax.numpy as jnp
from jax import lax
from jax.experimental import pallas as pl
from jax.experimental.pallas import tpu as pltpu
from jax.experimental.pallas import tpu_sc as plsc

_N = 10000
_D = 128
_H = 256
_BN_EPS = 1e-5

_W = 128
_NC = 2
_NS = 16
_NWORK = _NC * _NS
_ACC_ROWS = 10240
_ZROWS = _ACC_ROWS // _NS


def _sc_aggregate(x, src_p, dst_p, zeros, wpw):
    mesh = plsc.VectorSubcoreMesh(core_axis_name="c", subcore_axis_name="s")

    @functools.partial(
        pl.kernel,
        out_type=jax.ShapeDtypeStruct((_NC, _N, _D), jnp.float32),
        mesh=mesh,
        scratch_types=[
            pltpu.VMEM((_W,), jnp.int32),
            pltpu.VMEM((_W,), jnp.int32),
            pltpu.VMEM((_W, _D), jnp.float32),
            pltpu.VMEM_SHARED((_ACC_ROWS, _D), jnp.float32),
        ],
    )
    def agg_kernel(x_hbm, src_hbm, dst_hbm, z_hbm, out_hbm, sidx, didx, rows, acc):
        cid = lax.axis_index("c")
        sid = lax.axis_index("s")
        pltpu.sync_copy(z_hbm, acc.at[pl.ds(sid * _ZROWS, _ZROWS)])
        plsc.subcore_barrier()
        base = (cid * _NS + sid) * wpw

        @pl.loop(0, wpw)
        def _(i):
            w = (base + i) * _W
            pltpu.sync_copy(src_hbm.at[pl.ds(w, _W)], sidx)
            pltpu.sync_copy(dst_hbm.at[pl.ds(w, _W)], didx)
            pltpu.sync_copy(x_hbm.at[sidx], rows)
            pltpu.sync_copy(rows, acc.at[didx], add=True)

        plsc.subcore_barrier()
        rpw = 624
        pltpu.sync_copy(acc.at[pl.ds(sid * rpw, rpw)],
                        out_hbm.at[cid].at[pl.ds(sid * rpw, rpw)])

        @pl.when(sid == _NS - 1)
        def _():
            tail = _NS * rpw
            pltpu.sync_copy(acc.at[pl.ds(tail, _N - tail)],
                            out_hbm.at[cid].at[pl.ds(tail, _N - tail)])

    return agg_kernel(x, src_p, dst_p, zeros)


def _mlp_body(eps_ref, x_ref, agg_ref, w1_ref, b1_ref, g1_ref, be1_ref,
              w2_ref, b2_ref, g2_ref, be2_ref, o_ref):
    h = (1.0 + eps_ref[0]) * x_ref[...] + agg_ref[0] + agg_ref[1]
    t = jnp.dot(h, w1_ref[...], preferred_element_type=jnp.float32) + b1_ref[...]
    mu = jnp.mean(t, axis=0, keepdims=True)
    var = jnp.mean(jnp.square(t - mu), axis=0, keepdims=True)
    t = (t - mu) * lax.rsqrt(var + _BN_EPS) * g1_ref[...] + be1_ref[...]
    t = jnp.maximum(t, 0.0)
    u = jnp.dot(t, w2_ref[...], preferred_element_type=jnp.float32) + b2_ref[...]
    mu2 = jnp.mean(u, axis=0, keepdims=True)
    var2 = jnp.mean(jnp.square(u - mu2), axis=0, keepdims=True)
    u = (u - mu2) * lax.rsqrt(var2 + _BN_EPS) * g2_ref[...] + be2_ref[...]
    o_ref[...] = jnp.maximum(u, 0.0)


def _mlp(eps, x, aggpair, W1, b1, g1, be1, W2, b2, g2, be2):
    return pl.pallas_call(
        _mlp_body,
        out_shape=jax.ShapeDtypeStruct((_N, _D), jnp.float32),
        in_specs=[pl.BlockSpec(memory_space=pltpu.SMEM)]
                 + [pl.BlockSpec(memory_space=pltpu.VMEM)] * 10,
        out_specs=pl.BlockSpec(memory_space=pltpu.VMEM),
    )(eps, x, aggpair, W1, b1, g1, be1, W2, b2, g2, be2)


def kernel(x, edge_index, eps, W1, b1, g1, be1, W2, b2, g2, be2):
    E = edge_index.shape[1]
    wpw = -(-E // (_W * _NWORK))
    epad = _NWORK * wpw * _W
    pad = epad - E
    src = edge_index[0]
    dst = edge_index[1]
    if pad:
        ar = jnp.arange(pad, dtype=jnp.int32)
        src = jnp.concatenate([src, ar % _N])
        dst = jnp.concatenate([dst, _N + ar % (_ACC_ROWS - _N)])
    zeros = jnp.zeros((_ZROWS, _D), jnp.float32)
    aggpair = _sc_aggregate(x, src, dst, zeros, wpw)
    return _mlp(jnp.reshape(eps, (1,)), x, aggpair,
                W1, jnp.reshape(b1, (1, _H)), jnp.reshape(g1, (1, _H)),
                jnp.reshape(be1, (1, _H)),
                W2, jnp.reshape(b2, (1, _D)), jnp.reshape(g2, (1, _D)),
                jnp.reshape(be2, (1, _D)))


# speedup vs baseline: 10.6950x; 1.6307x over previous
import functools

import jax
import jax.numpy as jnp
from jax import lax
from jax.experimental import pallas as pl
from jax.experimental.pallas import tpu as pltpu
from jax.experimental.pallas import tpu_sc as plsc

_N = 10000
_D = 128
_H = 256
_BN_EPS = 1e-5

_W = 128
_NC = 2
_NS = 16
_NWORK = _NC * _NS
_ACC_ROWS = 10240
_ZROWS = _ACC_ROWS // _NS


def _sc_aggregate(x, src_p, dst_p, zeros, wpw):
    mesh = plsc.VectorSubcoreMesh(core_axis_name="c", subcore_axis_name="s")

    @functools.partial(
        pl.kernel,
        out_type=jax.ShapeDtypeStruct((_NC, _N, _D), jnp.float32),
        mesh=mesh,
        scratch_types=[
            pltpu.VMEM((wpw // 2, _W), jnp.int32),
            pltpu.VMEM((wpw // 2, _W), jnp.int32),
            pltpu.VMEM((_W, _D), jnp.float32),
            pltpu.VMEM((_W, _D), jnp.float32),
            pltpu.SemaphoreType.DMA,
            pltpu.SemaphoreType.DMA,
            pltpu.VMEM_SHARED((_ACC_ROWS, _D), jnp.float32),
        ],
    )
    def agg_kernel(x_hbm, src_hbm, dst_hbm, z_hbm, out_hbm,
                   sidx, didx, rows0, rows1, sem0, sem1, acc):
        cid = lax.axis_index("c")
        sid = lax.axis_index("s")
        wid = cid * _NS + sid
        hw = wpw // 2
        pltpu.sync_copy(z_hbm, acc.at[pl.ds(sid * _ZROWS, _ZROWS)])

        rows = (rows0, rows1)
        sems = (sem0, sem1)

        for half in range(2):
            pltpu.sync_copy(src_hbm.at[wid].at[pl.ds(half * hw, hw)], sidx)
            pltpu.sync_copy(dst_hbm.at[wid].at[pl.ds(half * hw, hw)], didx)
            pltpu.async_copy(x_hbm.at[sidx.at[0]], rows0, sem0)
            if half == 0:
                plsc.subcore_barrier()

            @pl.loop(0, hw, step=2)
            def _(i):
                for b in range(2):
                    w = i + b
                    pltpu.make_async_copy(x_hbm.at[sidx.at[w]], rows[b],
                                          sems[b]).wait()

                    @pl.when(w + 1 < hw)
                    def _():
                        pltpu.async_copy(x_hbm.at[sidx.at[w + 1]],
                                         rows[1 - b], sems[1 - b])

                    pltpu.sync_copy(rows[b], acc.at[didx.at[w]], add=True)

        plsc.subcore_barrier()
        rpw = 624
        pltpu.sync_copy(acc.at[pl.ds(sid * rpw, rpw)],
                        out_hbm.at[cid].at[pl.ds(sid * rpw, rpw)])

        @pl.when(sid == _NS - 1)
        def _():
            tail = _NS * rpw
            pltpu.sync_copy(acc.at[pl.ds(tail, _N - tail)],
                            out_hbm.at[cid].at[pl.ds(tail, _N - tail)])

    return agg_kernel(x, src_p, dst_p, zeros)


def _mlp_body(eps_ref, x_ref, agg_ref, w1_ref, b1_ref, g1_ref, be1_ref,
              w2_ref, b2_ref, g2_ref, be2_ref, o_ref):
    h = (1.0 + eps_ref[0]) * x_ref[...] + agg_ref[0] + agg_ref[1]
    t = jnp.dot(h, w1_ref[...], preferred_element_type=jnp.float32) + b1_ref[...]
    mu = jnp.mean(t, axis=0, keepdims=True)
    var = jnp.mean(jnp.square(t - mu), axis=0, keepdims=True)
    t = (t - mu) * lax.rsqrt(var + _BN_EPS) * g1_ref[...] + be1_ref[...]
    t = jnp.maximum(t, 0.0)
    u = jnp.dot(t, w2_ref[...], preferred_element_type=jnp.float32) + b2_ref[...]
    mu2 = jnp.mean(u, axis=0, keepdims=True)
    var2 = jnp.mean(jnp.square(u - mu2), axis=0, keepdims=True)
    u = (u - mu2) * lax.rsqrt(var2 + _BN_EPS) * g2_ref[...] + be2_ref[...]
    o_ref[...] = jnp.maximum(u, 0.0)


def _mlp(eps, x, aggpair, W1, b1, g1, be1, W2, b2, g2, be2):
    return pl.pallas_call(
        _mlp_body,
        out_shape=jax.ShapeDtypeStruct((_N, _D), jnp.float32),
        in_specs=[pl.BlockSpec(memory_space=pltpu.SMEM)]
                 + [pl.BlockSpec(memory_space=pltpu.VMEM)] * 10,
        out_specs=pl.BlockSpec(memory_space=pltpu.VMEM),
    )(eps, x, aggpair, W1, b1, g1, be1, W2, b2, g2, be2)


def kernel(x, edge_index, eps, W1, b1, g1, be1, W2, b2, g2, be2):
    E = edge_index.shape[1]
    wpw = -(-E // (_W * _NWORK))
    wpw = -(-wpw // 4) * 4
    epad = _NWORK * wpw * _W
    pad = epad - E
    src = edge_index[0]
    dst = edge_index[1]
    if pad:
        ar = jnp.arange(pad, dtype=jnp.int32)
        src = jnp.concatenate([src, ar % _N])
        dst = jnp.concatenate([dst, _N + ar % (_ACC_ROWS - _N)])
    src = jnp.reshape(src, (_NWORK, wpw, _W))
    dst = jnp.reshape(dst, (_NWORK, wpw, _W))
    zeros = jnp.zeros((_ZROWS, _D), jnp.float32)
    aggpair = _sc_aggregate(x, src, dst, zeros, wpw)
    return _mlp(jnp.reshape(eps, (1,)), x, aggpair,
                W1, jnp.reshape(b1, (1, _H)), jnp.reshape(g1, (1, _H)),
                jnp.reshape(be1, (1, _H)),
                W2, jnp.reshape(b2, (1, _D)), jnp.reshape(g2, (1, _D)),
                jnp.reshape(be2, (1, _D)))


# speedup vs baseline: 12.3688x; 1.1565x over previous
import functools

import jax
import jax.numpy as jnp
from jax import lax
from jax.experimental import pallas as pl
from jax.experimental.pallas import tpu as pltpu
from jax.experimental.pallas import tpu_sc as plsc

_N = 10000
_D = 128
_H = 256
_BN_EPS = 1e-5

_W = 128
_NC = 2
_NS = 16
_NWORK = _NC * _NS
_ACC_ROWS = 10240
_ZROWS = _ACC_ROWS // _NS


def _sc_aggregate(x, src_p, dst_p, zeros, wpw):
    mesh = plsc.VectorSubcoreMesh(core_axis_name="c", subcore_axis_name="s")

    @functools.partial(
        pl.kernel,
        out_type=jax.ShapeDtypeStruct((_NC, _N, _D), jnp.float32),
        mesh=mesh,
        scratch_types=(
            [pltpu.VMEM((_W,), jnp.int32)] * 4
            + [pltpu.VMEM((_W,), jnp.int32)] * 4
            + [pltpu.VMEM((_W, _D), jnp.float32)] * 2
            + [pltpu.SemaphoreType.DMA] * 8
            + [pltpu.VMEM_SHARED((_ACC_ROWS, _D), jnp.float32)]
        ),
    )
    def agg_kernel(x_hbm, src_hbm, dst_hbm, z_hbm, out_hbm,
                   si0, si1, si2, si3, di0, di1, di2, di3, rows0, rows1,
                   is0, is1, is2, is3, gs0, gs1, ss0, ss1, acc):
        cid = lax.axis_index("c")
        sid = lax.axis_index("s")
        wid = cid * _NS + sid
        sidx = (si0, si1, si2, si3)
        didx = (di0, di1, di2, di3)
        isem = (is0, is1, is2, is3)
        rows = (rows0, rows1)
        gsem = (gs0, gs1)
        ssem = (ss0, ss1)

        def idx_start(i, slot):
            pltpu.async_copy(src_hbm.at[wid].at[i], sidx[slot], isem[slot])
            pltpu.async_copy(dst_hbm.at[wid].at[i], didx[slot], isem[slot])

        def idx_wait(i, slot):
            pltpu.make_async_copy(src_hbm.at[wid].at[i], sidx[slot],
                                  isem[slot]).wait()
            pltpu.make_async_copy(dst_hbm.at[wid].at[i], didx[slot],
                                  isem[slot]).wait()

        def gather_start(i, slot):
            pltpu.async_copy(x_hbm.at[sidx[slot % 4]], rows[slot % 2],
                             gsem[slot % 2])

        def gather_wait(i, slot):
            pltpu.make_async_copy(x_hbm.at[sidx[slot % 4]], rows[slot % 2],
                                  gsem[slot % 2]).wait()

        pltpu.sync_copy(z_hbm, acc.at[pl.ds(sid * _ZROWS, _ZROWS)])
        idx_start(0, 0)
        idx_start(1, 1)
        idx_wait(0, 0)
        gather_start(0, 0)
        plsc.subcore_barrier()

        @pl.loop(0, wpw, step=4)
        def _(base):
            for k in range(4):
                i = base + k
                @pl.when(i + 2 < wpw)
                def _():
                    idx_start(i + 2, (k + 2) % 4)

                @pl.when(i + 1 < wpw)
                def _():
                    idx_wait(i + 1, (k + 1) % 4)

                    @pl.when(i >= 1)
                    def _():
                        pltpu.make_async_copy(
                            rows[(k + 1) % 2],
                            acc.at[didx[(k + 3) % 4]],
                            ssem[(k + 1) % 2]).wait()

                    gather_start(i + 1, k + 1)

                gather_wait(i, k)
                pltpu.async_copy(rows[k % 2], acc.at[didx[k % 4]],
                                 ssem[k % 2], add=True)

        pltpu.make_async_copy(rows[0], acc.at[didx[(wpw - 2) % 4]],
                              ssem[0]).wait()
        pltpu.make_async_copy(rows[1], acc.at[didx[(wpw - 1) % 4]],
                              ssem[1]).wait()

        plsc.subcore_barrier()
        rpw = 624
        pltpu.sync_copy(acc.at[pl.ds(sid * rpw, rpw)],
                        out_hbm.at[cid].at[pl.ds(sid * rpw, rpw)])

        @pl.when(sid == _NS - 1)
        def _():
            tail = _NS * rpw
            pltpu.sync_copy(acc.at[pl.ds(tail, _N - tail)],
                            out_hbm.at[cid].at[pl.ds(tail, _N - tail)])

    return agg_kernel(x, src_p, dst_p, zeros)


def _mlp_body(eps_ref, x_ref, agg_ref, w1_ref, b1_ref, g1_ref, be1_ref,
              w2_ref, b2_ref, g2_ref, be2_ref, o_ref):
    h = (1.0 + eps_ref[0]) * x_ref[...] + agg_ref[0] + agg_ref[1]
    t = jnp.dot(h, w1_ref[...], preferred_element_type=jnp.float32) + b1_ref[...]
    mu = jnp.mean(t, axis=0, keepdims=True)
    var = jnp.mean(jnp.square(t - mu), axis=0, keepdims=True)
    t = (t - mu) * lax.rsqrt(var + _BN_EPS) * g1_ref[...] + be1_ref[...]
    t = jnp.maximum(t, 0.0)
    u = jnp.dot(t, w2_ref[...], preferred_element_type=jnp.float32) + b2_ref[...]
    mu2 = jnp.mean(u, axis=0, keepdims=True)
    var2 = jnp.mean(jnp.square(u - mu2), axis=0, keepdims=True)
    u = (u - mu2) * lax.rsqrt(var2 + _BN_EPS) * g2_ref[...] + be2_ref[...]
    o_ref[...] = jnp.maximum(u, 0.0)


def _mlp(eps, x, aggpair, W1, b1, g1, be1, W2, b2, g2, be2):
    return pl.pallas_call(
        _mlp_body,
        out_shape=jax.ShapeDtypeStruct((_N, _D), jnp.float32),
        in_specs=[pl.BlockSpec(memory_space=pltpu.SMEM)]
                 + [pl.BlockSpec(memory_space=pltpu.VMEM)] * 10,
        out_specs=pl.BlockSpec(memory_space=pltpu.VMEM),
    )(eps, x, aggpair, W1, b1, g1, be1, W2, b2, g2, be2)


def kernel(x, edge_index, eps, W1, b1, g1, be1, W2, b2, g2, be2):
    E = edge_index.shape[1]
    wpw = -(-E // (_W * _NWORK))
    wpw = -(-wpw // 4) * 4
    epad = _NWORK * wpw * _W
    pad = epad - E
    src = edge_index[0]
    dst = edge_index[1]
    if pad:
        ar = jnp.arange(pad, dtype=jnp.int32)
        src = jnp.concatenate([src, ar % _N])
        dst = jnp.concatenate([dst, _N + ar % (_ACC_ROWS - _N)])
    src = jnp.reshape(src, (_NWORK, wpw, _W))
    dst = jnp.reshape(dst, (_NWORK, wpw, _W))
    zeros = jnp.zeros((_ZROWS, _D), jnp.float32)
    aggpair = _sc_aggregate(x, src, dst, zeros, wpw)
    return _mlp(jnp.reshape(eps, (1,)), x, aggpair,
                W1, jnp.reshape(b1, (1, _H)), jnp.reshape(g1, (1, _H)),
                jnp.reshape(be1, (1, _H)),
                W2, jnp.reshape(b2, (1, _D)), jnp.reshape(g2, (1, _D)),
                jnp.reshape(be2, (1, _D)))


# speedup vs baseline: 12.4437x; 1.0061x over previous
import functools

import jax
import jax.numpy as jnp
from jax import lax
from jax.experimental import pallas as pl
from jax.experimental.pallas import tpu as pltpu
from jax.experimental.pallas import tpu_sc as plsc

_N = 10000
_D = 128
_H = 256
_BN_EPS = 1e-5

_W = 128
_NC = 2
_NS = 16
_NWORK = _NC * _NS
_ACC_ROWS = 10240
_ZROWS = _ACC_ROWS // _NS


def _sc_aggregate(x, src, dst, zeros, nwin):
    niter = -(-nwin // _NWORK)
    niter = -(-niter // 4) * 4
    mesh = plsc.VectorSubcoreMesh(core_axis_name="c", subcore_axis_name="s")

    @functools.partial(
        pl.kernel,
        out_type=jax.ShapeDtypeStruct((_NC, _N, _D), jnp.float32),
        mesh=mesh,
        scratch_types=(
            [pltpu.VMEM((_W,), jnp.int32)] * 4
            + [pltpu.VMEM((_W,), jnp.int32)] * 4
            + [pltpu.VMEM((_W, _D), jnp.float32)] * 2
            + [pltpu.SemaphoreType.DMA] * 8
            + [pltpu.VMEM_SHARED((_ACC_ROWS, _D), jnp.float32)]
        ),
    )
    def agg_kernel(x_hbm, src_hbm, dst_hbm, z_hbm, out_hbm,
                   si0, si1, si2, si3, di0, di1, di2, di3, rows0, rows1,
                   is0, is1, is2, is3, gs0, gs1, ss0, ss1, acc):
        cid = lax.axis_index("c")
        sid = lax.axis_index("s")
        wid = cid * _NS + sid
        sidx = (si0, si1, si2, si3)
        didx = (di0, di1, di2, di3)
        isem = (is0, is1, is2, is3)
        rows = (rows0, rows1)
        gsem = (gs0, gs1)
        ssem = (ss0, ss1)

        def win(i):
            return (wid + i * _NWORK) * _W

        def idx_start(i, slot):
            pltpu.async_copy(src_hbm.at[pl.ds(win(i), _W)], sidx[slot],
                             isem[slot])
            pltpu.async_copy(dst_hbm.at[pl.ds(win(i), _W)], didx[slot],
                             isem[slot])

        def idx_wait(i, slot):
            pltpu.make_async_copy(src_hbm.at[pl.ds(win(i), _W)], sidx[slot],
                                  isem[slot]).wait()
            pltpu.make_async_copy(dst_hbm.at[pl.ds(win(i), _W)], didx[slot],
                                  isem[slot]).wait()

        def gather_start(slot):
            pltpu.async_copy(x_hbm.at[sidx[slot % 4]], rows[slot % 2],
                             gsem[slot % 2])

        def gather_wait(slot):
            pltpu.make_async_copy(x_hbm.at[sidx[slot % 4]], rows[slot % 2],
                                  gsem[slot % 2]).wait()

        pltpu.sync_copy(z_hbm, acc.at[pl.ds(sid * _ZROWS, _ZROWS)])
        idx_start(0, 0)
        idx_start(1, 1)
        idx_wait(0, 0)
        gather_start(0)
        plsc.subcore_barrier()

        nvalid = (nwin - wid + _NWORK - 1) // _NWORK

        @pl.loop(0, niter, step=4)
        def _(base):
            for k in range(4):
                i = base + k
                @pl.when(i + 2 < nvalid)
                def _():
                    idx_start(i + 2, (k + 2) % 4)

                @pl.when(i + 1 < nvalid)
                def _():
                    idx_wait(i + 1, (k + 1) % 4)

                    @pl.when(i >= 1)
                    def _():
                        pltpu.make_async_copy(
                            rows[(k + 1) % 2],
                            acc.at[didx[(k + 3) % 4]],
                            ssem[(k + 1) % 2]).wait()

                    gather_start(k + 1)

                @pl.when(i < nvalid)
                def _():
                    gather_wait(k)
                    pltpu.async_copy(rows[k % 2], acc.at[didx[k % 4]],
                                     ssem[k % 2], add=True)

        pltpu.make_async_copy(rows[0], acc.at[didx[0]], ssem[0]).wait()
        pltpu.make_async_copy(rows[1], acc.at[didx[1]], ssem[1]).wait()

        plsc.subcore_barrier()
        rpw = 624
        pltpu.sync_copy(acc.at[pl.ds(sid * rpw, rpw)],
                        out_hbm.at[cid].at[pl.ds(sid * rpw, rpw)])

        @pl.when(sid == _NS - 1)
        def _():
            tail = _NS * rpw
            pltpu.sync_copy(acc.at[pl.ds(tail, _N - tail)],
                            out_hbm.at[cid].at[pl.ds(tail, _N - tail)])

    return agg_kernel(x, src, dst, zeros)


def _mlp_body(eps_ref, x_ref, agg_ref, w1_ref, b1_ref, g1_ref, be1_ref,
              w2_ref, b2_ref, g2_ref, be2_ref, o_ref):
    h = (1.0 + eps_ref[0]) * x_ref[...] + agg_ref[0] + agg_ref[1]
    t = jnp.dot(h.astype(jnp.bfloat16), w1_ref[...].astype(jnp.bfloat16),
                preferred_element_type=jnp.float32) + b1_ref[...]
    mu = jnp.mean(t, axis=0, keepdims=True)
    var = jnp.mean(jnp.square(t - mu), axis=0, keepdims=True)
    t = (t - mu) * lax.rsqrt(var + _BN_EPS) * g1_ref[...] + be1_ref[...]
    t = jnp.maximum(t, 0.0)
    u = jnp.dot(t.astype(jnp.bfloat16), w2_ref[...].astype(jnp.bfloat16),
                preferred_element_type=jnp.float32) + b2_ref[...]
    mu2 = jnp.mean(u, axis=0, keepdims=True)
    var2 = jnp.mean(jnp.square(u - mu2), axis=0, keepdims=True)
    u = (u - mu2) * lax.rsqrt(var2 + _BN_EPS) * g2_ref[...] + be2_ref[...]
    o_ref[...] = jnp.maximum(u, 0.0)


def _mlp(eps, x, aggpair, W1, b1, g1, be1, W2, b2, g2, be2):
    return pl.pallas_call(
        _mlp_body,
        out_shape=jax.ShapeDtypeStruct((_N, _D), jnp.float32),
        in_specs=[pl.BlockSpec(memory_space=pltpu.SMEM)]
                 + [pl.BlockSpec(memory_space=pltpu.VMEM)] * 10,
        out_specs=pl.BlockSpec(memory_space=pltpu.VMEM),
    )(eps, x, aggpair, W1, b1, g1, be1, W2, b2, g2, be2)


def kernel(x, edge_index, eps, W1, b1, g1, be1, W2, b2, g2, be2):
    E = edge_index.shape[1]
    src = edge_index[0]
    dst = edge_index[1]
    rem = E % _W
    if rem:
        pad = _W - rem
        ar = jnp.arange(pad, dtype=jnp.int32)
        src = jnp.concatenate([src, ar % _N])
        dst = jnp.concatenate([dst, _N + ar % (_ACC_ROWS - _N)])
    nwin = (E + _W - 1) // _W
    zeros = jnp.zeros((_ZROWS, _D), jnp.float32)
    aggpair = _sc_aggregate(x, src, dst, zeros, nwin)
    return _mlp(jnp.reshape(eps, (1,)), x, aggpair,
                W1, jnp.reshape(b1, (1, _H)), jnp.reshape(g1, (1, _H)),
                jnp.reshape(be1, (1, _H)),
                W2, jnp.reshape(b2, (1, _D)), jnp.reshape(g2, (1, _D)),
                jnp.reshape(be2, (1, _D)))


# speedup vs baseline: 13.5911x; 1.0922x over previous
import functools

import jax
import jax.numpy as jnp
from jax import lax
from jax.experimental import pallas as pl
from jax.experimental.pallas import tpu as pltpu
from jax.experimental.pallas import tpu_sc as plsc

_N = 10000
_D = 128
_H = 256
_BN_EPS = 1e-5

_W = 128
_NC = 2
_NS = 16
_NWORK = _NC * _NS
_ACC_ROWS = 10240
_ZROWS = _ACC_ROWS // _NS


def _sc_aggregate(x, edges, zeros, nwin):
    niter = -(-nwin // _NWORK)
    niter = -(-niter // 4) * 4
    mesh = plsc.VectorSubcoreMesh(core_axis_name="c", subcore_axis_name="s")

    @functools.partial(
        pl.kernel,
        out_type=jax.ShapeDtypeStruct((_NC, _N, _D), jnp.float32),
        mesh=mesh,
        scratch_types=(
            [pltpu.VMEM((2, _W), jnp.int32)] * 4
            + [pltpu.VMEM((_W, _D), jnp.float32)] * 2
            + [pltpu.SemaphoreType.DMA] * 8
            + [pltpu.VMEM_SHARED((_ACC_ROWS, _D), jnp.float32)]
        ),
    )
    def agg_kernel(x_hbm, e_hbm, z_hbm, out_hbm,
                   ib0, ib1, ib2, ib3, rows0, rows1,
                   is0, is1, is2, is3, gs0, gs1, ss0, ss1, acc):
        cid = lax.axis_index("c")
        sid = lax.axis_index("s")
        wid = cid * _NS + sid
        ibuf = (ib0, ib1, ib2, ib3)
        isem = (is0, is1, is2, is3)
        rows = (rows0, rows1)
        gsem = (gs0, gs1)
        ssem = (ss0, ss1)

        def win(i):
            return (wid + i * _NWORK) * _W

        def idx_start(i, slot):
            pltpu.async_copy(e_hbm.at[:, pl.ds(win(i), _W)], ibuf[slot],
                             isem[slot])

        def idx_wait(i, slot):
            pltpu.make_async_copy(e_hbm.at[:, pl.ds(win(i), _W)], ibuf[slot],
                                  isem[slot]).wait()

        def gather_start(slot):
            pltpu.async_copy(x_hbm.at[ibuf[slot % 4].at[0]], rows[slot % 2],
                             gsem[slot % 2])

        def gather_wait(slot):
            pltpu.make_async_copy(x_hbm.at[ibuf[slot % 4].at[0]],
                                  rows[slot % 2], gsem[slot % 2]).wait()

        pltpu.sync_copy(z_hbm, acc.at[pl.ds(sid * _ZROWS, _ZROWS)])
        idx_start(0, 0)
        idx_start(1, 1)
        idx_wait(0, 0)
        gather_start(0)
        plsc.subcore_barrier()

        nvalid = (nwin - wid + _NWORK - 1) // _NWORK

        @pl.loop(0, niter, step=4)
        def _(base):
            for k in range(4):
                i = base + k
                @pl.when(i + 2 < nvalid)
                def _():
                    idx_start(i + 2, (k + 2) % 4)

                @pl.when(i + 1 < nvalid)
                def _():
                    idx_wait(i + 1, (k + 1) % 4)

                    @pl.when(i >= 1)
                    def _():
                        pltpu.make_async_copy(
                            rows[(k + 1) % 2],
                            acc.at[ibuf[(k + 3) % 4].at[1]],
                            ssem[(k + 1) % 2]).wait()

                    gather_start(k + 1)

                @pl.when(i < nvalid)
                def _():
                    gather_wait(k)
                    pltpu.async_copy(rows[k % 2], acc.at[ibuf[k % 4].at[1]],
                                     ssem[k % 2], add=True)

        pltpu.make_async_copy(rows[0], acc.at[ibuf[0].at[1]], ssem[0]).wait()
        pltpu.make_async_copy(rows[1], acc.at[ibuf[1].at[1]], ssem[1]).wait()

        plsc.subcore_barrier()
        rpw = 624
        pltpu.sync_copy(acc.at[pl.ds(sid * rpw, rpw)],
                        out_hbm.at[cid].at[pl.ds(sid * rpw, rpw)])

        @pl.when(sid == _NS - 1)
        def _():
            tail = _NS * rpw
            pltpu.sync_copy(acc.at[pl.ds(tail, _N - tail)],
                            out_hbm.at[cid].at[pl.ds(tail, _N - tail)])

    return agg_kernel(x, edges, zeros)


def _mlp_body(eps_ref, x_ref, agg_ref, w1_ref, b1_ref, g1_ref, be1_ref,
              w2_ref, b2_ref, g2_ref, be2_ref, o_ref):
    h = (1.0 + eps_ref[0]) * x_ref[...] + agg_ref[0] + agg_ref[1]
    t = jnp.dot(h.astype(jnp.bfloat16), w1_ref[...].astype(jnp.bfloat16),
                preferred_element_type=jnp.float32) + b1_ref[...]
    mu = jnp.mean(t, axis=0, keepdims=True)
    var = jnp.mean(jnp.square(t - mu), axis=0, keepdims=True)
    t = (t - mu) * lax.rsqrt(var + _BN_EPS) * g1_ref[...] + be1_ref[...]
    t = jnp.maximum(t, 0.0)
    u = jnp.dot(t.astype(jnp.bfloat16), w2_ref[...].astype(jnp.bfloat16),
                preferred_element_type=jnp.float32) + b2_ref[...]
    mu2 = jnp.mean(u, axis=0, keepdims=True)
    var2 = jnp.mean(jnp.square(u - mu2), axis=0, keepdims=True)
    u = (u - mu2) * lax.rsqrt(var2 + _BN_EPS) * g2_ref[...] + be2_ref[...]
    o_ref[...] = jnp.maximum(u, 0.0)


def _mlp(eps, x, aggpair, W1, b1, g1, be1, W2, b2, g2, be2):
    return pl.pallas_call(
        _mlp_body,
        out_shape=jax.ShapeDtypeStruct((_N, _D), jnp.float32),
        in_specs=[pl.BlockSpec(memory_space=pltpu.SMEM)]
                 + [pl.BlockSpec(memory_space=pltpu.VMEM)] * 10,
        out_specs=pl.BlockSpec(memory_space=pltpu.VMEM),
    )(eps, x, aggpair, W1, b1, g1, be1, W2, b2, g2, be2)


def kernel(x, edge_index, eps, W1, b1, g1, be1, W2, b2, g2, be2):
    E = edge_index.shape[1]
    rem = E % _W
    edges = edge_index
    if rem:
        pad = _W - rem
        ar = jnp.arange(pad, dtype=jnp.int32)
        edges = jnp.concatenate(
            [edge_index,
             jnp.stack([ar % _N, _N + ar % (_ACC_ROWS - _N)])], axis=1)
    nwin = (E + _W - 1) // _W
    zeros = jnp.zeros((_ZROWS, _D), jnp.float32)
    aggpair = _sc_aggregate(x, edges, zeros, nwin)
    return _mlp(jnp.reshape(eps, (1,)), x, aggpair,
                W1, jnp.reshape(b1, (1, _H)), jnp.reshape(g1, (1, _H)),
                jnp.reshape(be1, (1, _H)),
                W2, jnp.reshape(b2, (1, _D)), jnp.reshape(g2, (1, _D)),
                jnp.reshape(be2, (1, _D)))


# speedup vs baseline: 13.5967x; 1.0004x over previous
import functools

import jax
import jax.numpy as jnp
from jax import lax
from jax.experimental import pallas as pl
from jax.experimental.pallas import tpu as pltpu
from jax.experimental.pallas import tpu_sc as plsc

_N = 10000
_D = 128
_H = 256
_BN_EPS = 1e-5

_W = 128
_NC = 2
_NS = 16
_NWORK = _NC * _NS
_ACC_ROWS = 10240
_ZROWS = _ACC_ROWS // _NS


def _sc_aggregate(x, edges, zeros, nwin):
    niter = -(-nwin // _NWORK)
    niter = -(-niter // 4) * 4
    mesh = plsc.VectorSubcoreMesh(core_axis_name="c", subcore_axis_name="s")

    @functools.partial(
        pl.kernel,
        out_type=jax.ShapeDtypeStruct((_NC, _N, _D), jnp.float32),
        mesh=mesh,
        scratch_types=(
            [pltpu.VMEM((2, _W), jnp.int32)] * 4
            + [pltpu.VMEM((_W, _D), jnp.float32)] * 2
            + [pltpu.SemaphoreType.DMA] * 8
            + [pltpu.VMEM_SHARED((_ACC_ROWS, _D), jnp.float32)]
        ),
    )
    def agg_kernel(x_hbm, e_hbm, z_hbm, out_hbm,
                   ib0, ib1, ib2, ib3, rows0, rows1,
                   is0, is1, is2, is3, gs0, gs1, ss0, ss1, acc):
        cid = lax.axis_index("c")
        sid = lax.axis_index("s")
        wid = cid * _NS + sid
        ibuf = (ib0, ib1, ib2, ib3)
        isem = (is0, is1, is2, is3)
        rows = (rows0, rows1)
        gsem = (gs0, gs1)
        ssem = (ss0, ss1)

        def win(i):
            return (wid + i * _NWORK) * _W

        def idx_start(i, slot):
            pltpu.async_copy(e_hbm.at[:, pl.ds(win(i), _W)], ibuf[slot],
                             isem[slot])

        def idx_wait(i, slot):
            pltpu.make_async_copy(e_hbm.at[:, pl.ds(win(i), _W)], ibuf[slot],
                                  isem[slot]).wait()

        def gather_start(slot):
            pltpu.async_copy(x_hbm.at[ibuf[slot % 4].at[0]], rows[slot % 2],
                             gsem[slot % 2])

        def gather_wait(slot):
            pltpu.make_async_copy(x_hbm.at[ibuf[slot % 4].at[0]],
                                  rows[slot % 2], gsem[slot % 2]).wait()

        pltpu.sync_copy(z_hbm, acc.at[pl.ds(sid * _ZROWS, _ZROWS)])
        idx_start(0, 0)
        idx_start(1, 1)
        idx_wait(0, 0)
        gather_start(0)
        plsc.subcore_barrier()

        nvalid = (nwin - wid + _NWORK - 1) // _NWORK

        @pl.loop(0, niter, step=4)
        def _(base):
            for k in range(4):
                i = base + k
                @pl.when(i + 2 < nvalid)
                def _():
                    idx_start(i + 2, (k + 2) % 4)

                @pl.when(i + 1 < nvalid)
                def _():
                    idx_wait(i + 1, (k + 1) % 4)

                    @pl.when(i >= 1)
                    def _():
                        pltpu.make_async_copy(
                            rows[(k + 1) % 2],
                            acc.at[ibuf[(k + 3) % 4].at[1]],
                            ssem[(k + 1) % 2]).wait()

                    gather_start(k + 1)

                @pl.when(i < nvalid)
                def _():
                    gather_wait(k)
                    pltpu.async_copy(rows[k % 2], acc.at[ibuf[k % 4].at[1]],
                                     ssem[k % 2], add=True)

        pltpu.make_async_copy(rows[0], acc.at[ibuf[0].at[1]], ssem[0]).wait()
        pltpu.make_async_copy(rows[1], acc.at[ibuf[1].at[1]], ssem[1]).wait()

        plsc.subcore_barrier()
        rpw = 624
        pltpu.sync_copy(acc.at[pl.ds(sid * rpw, rpw)],
                        out_hbm.at[cid].at[pl.ds(sid * rpw, rpw)])

        @pl.when(sid == _NS - 1)
        def _():
            tail = _NS * rpw
            pltpu.sync_copy(acc.at[pl.ds(tail, _N - tail)],
                            out_hbm.at[cid].at[pl.ds(tail, _N - tail)])

    return agg_kernel(x, edges, zeros)


def _bn_coeffs(tb, t2b, g, be):
    ones = jnp.ones((1, tb.shape[0]), jnp.bfloat16)
    s1 = jnp.dot(ones, tb, preferred_element_type=jnp.float32)
    s2 = jnp.dot(ones, t2b, preferred_element_type=jnp.float32)
    inv_n = 1.0 / tb.shape[0]
    mu = s1 * inv_n
    var = s2 * inv_n - mu * mu
    a = g * lax.rsqrt(var + _BN_EPS)
    return a, be - mu * a


def _mlp_body(eps_ref, x_ref, agg_ref, w1_ref, b1_ref, g1_ref, be1_ref,
              w2_ref, b2_ref, g2_ref, be2_ref, o_ref):
    h = (1.0 + eps_ref[0]) * x_ref[...] + agg_ref[0] + agg_ref[1]
    t = jnp.dot(h.astype(jnp.bfloat16), w1_ref[...].astype(jnp.bfloat16),
                preferred_element_type=jnp.float32)
    tb = t.astype(jnp.bfloat16)
    a1, c1 = _bn_coeffs(tb, tb * tb, g1_ref[...], be1_ref[...])
    t = jnp.maximum(t * a1 + c1, 0.0)
    u = jnp.dot(t.astype(jnp.bfloat16), w2_ref[...].astype(jnp.bfloat16),
                preferred_element_type=jnp.float32)
    ub = u.astype(jnp.bfloat16)
    a2, c2 = _bn_coeffs(ub, ub * ub, g2_ref[...], be2_ref[...])
    o_ref[...] = jnp.maximum(u * a2 + c2, 0.0)


def _mlp(eps, x, aggpair, W1, b1, g1, be1, W2, b2, g2, be2):
    return pl.pallas_call(
        _mlp_body,
        out_shape=jax.ShapeDtypeStruct((_N, _D), jnp.float32),
        in_specs=[pl.BlockSpec(memory_space=pltpu.SMEM)]
                 + [pl.BlockSpec(memory_space=pltpu.VMEM)] * 10,
        out_specs=pl.BlockSpec(memory_space=pltpu.VMEM),
    )(eps, x, aggpair, W1, b1, g1, be1, W2, b2, g2, be2)


def kernel(x, edge_index, eps, W1, b1, g1, be1, W2, b2, g2, be2):
    E = edge_index.shape[1]
    rem = E % _W
    edges = edge_index
    if rem:
        pad = _W - rem
        ar = jnp.arange(pad, dtype=jnp.int32)
        edges = jnp.concatenate(
            [edge_index,
             jnp.stack([ar % _N, _N + ar % (_ACC_ROWS - _N)])], axis=1)
    nwin = (E + _W - 1) // _W
    zeros = jnp.zeros((_ZROWS, _D), jnp.float32)
    aggpair = _sc_aggregate(x, edges, zeros, nwin)
    return _mlp(jnp.reshape(eps, (1,)), x, aggpair,
                W1, jnp.reshape(b1, (1, _H)), jnp.reshape(g1, (1, _H)),
                jnp.reshape(be1, (1, _H)),
                W2, jnp.reshape(b2, (1, _D)), jnp.reshape(g2, (1, _D)),
                jnp.reshape(be2, (1, _D)))
